# baseline (speedup 1.0000x reference)
import jax
import jax.numpy as jnp
from jax.experimental import pallas as pl

_NUM_ENT_PAIRING = 3
_NUM_PRED_EDGES = 1
_NUM_MAX_REL_PRED = 2048


def _copy_kernel(x_ref, o_ref):
    o_ref[...] = x_ref[...]


def _per_batch(rel_logits, hidx, oidx, ent_score):
    prob = jax.nn.sigmoid(rel_logits)
    sub_match = jax.nn.softmax(hidx, axis=-1)
    obj_match = jax.nn.softmax(oidx, axis=-1)
    k = min(_NUM_ENT_PAIRING, sub_match.shape[-1])
    _, sub_ids = jax.lax.top_k(sub_match, k)
    _, obj_ids = jax.lax.top_k(obj_match, k)
    Nq, C = prob.shape
    topk = Nq * _NUM_PRED_EDGES
    flat = prob.reshape(-1)
    vals, idxs = jax.lax.top_k(flat, topk)
    total_pred_idx = idxs // C
    pred_rel_labels = idxs % C + 1
    sub_ids_sel = sub_ids[total_pred_idx]
    obj_ids_sel = obj_ids[total_pred_idx]
    total_pred_idx_e = jnp.repeat(total_pred_idx[:, None], k, axis=1).reshape(-1)
    prob_e = jnp.repeat(vals[:, None], k, axis=1).reshape(-1)
    labels_e = jnp.repeat(pred_rel_labels[:, None], k, axis=1).reshape(-1)
    sub_flat = sub_ids_sel.reshape(-1)
    obj_flat = obj_ids_sel.reshape(-1)
    trp_scores = prob_e * ent_score[sub_flat] * ent_score[obj_flat]
    pred_rel_triplet = jnp.stack(
        [
            sub_flat.astype(jnp.float32),
            obj_flat.astype(jnp.float32),
            labels_e.astype(jnp.float32),
            trp_scores,
            prob_e,
            total_pred_idx_e.astype(jnp.float32),
        ],
        axis=1,
    )
    non_self = obj_flat != sub_flat
    masked = jnp.where(non_self, trp_scores, -jnp.inf)
    M = min(_NUM_MAX_REL_PRED, masked.shape[0])
    _, top_rel_idx = jax.lax.top_k(masked, M)
    return pred_rel_triplet[top_rel_idx]


def kernel(pred_rel_logits, pred_hidx, pred_oidx, ent_scores, ent_boxes, target_sizes):
    B = pred_rel_logits.shape[0]
    outs = []
    for b in range(B):
        outs.append(_per_batch(pred_rel_logits[b], pred_hidx[b], pred_oidx[b], ent_scores[b]))
    out = jnp.stack(outs, axis=0)
    return pl.pallas_call(
        _copy_kernel,
        out_shape=jax.ShapeDtypeStruct(out.shape, out.dtype),
    )(out)


# speedup vs baseline: 7.6238x; 7.6238x over previous
import functools

import jax
import jax.numpy as jnp
from jax import lax
from jax.experimental import pallas as pl
from jax.experimental.pallas import tpu as pltpu
from jax.experimental.pallas import tpu_sc as plsc

L = 16
NQ, C, NENT = 5000, 51, 300
NFLAT = 256000
SHARD = NFLAT // 16
SEG = SHARD // L
NROWP = 5120
RPT = NROWP // 16
CPAD = 304
ENTP = 512
K1 = 5000
N1 = 5008
N1P = 5024
SEG2 = N1 // L
NT = 15008
SEG3 = NT // L
K2 = 2048
SEG4 = K2 // L

AI_IDS_SUB = 0
AI_IDS_OBJ = 960
AI_KV = 1920
AI_H2D = 17920
AI_STGK = 22016
AI_STGV = 38016
AI_STGT = 54016
AI_IOTA = 70016
AI_HISTM = 70272
AI_ZERO = 70528
AI_END = 70784
SO_SK = 0
SO_SV = 5024
SO_KB = 10048
SO_VB = 15072
SO_SUB = 20096
SO_OBJ = 35456
SO_TK = 50816
SO_K2A = 65824
SO_V2A = 67888
SO_K2B = 70784
SO_V2B = 72848
SO_H32A = 74912
SO_H32B = 75424
SO_H2D5 = 75936
SO_HM5 = 80032
AI_TOTAL = 80288

AF_ROW0 = 0
AF_ROW1 = 2432
AF_PROBV = 4864
AF_ENT = 20864
AF_OUT = 21376
AF_TOTAL = 33664

SP_IDS = 0
SP_SURVK = 61440
SP_SURVI = 71488
SP_CNTD = 81536
SP_CNTE = 82048
SP_HIST = 82560
SP_TOTAL = 84608

NEG_INF = float("-inf")


def _i32(v):
    return jnp.full((L,), v, jnp.int32)


def _suffix_find(ai, hm_base, need, lanes):
    def body(t, car):
        carry, beta, sbeta = car
        jj = 15 - t
        h = ai[pl.ds(hm_base + jj * 16, 16)]
        tot = jnp.sum(h)
        cs = plsc.cumsum(h)
        s_vec = carry + (tot - cs)
        m = (s_vec < need) & (s_vec + h >= need)
        bidx = jj * 16 + lanes
        beta = jnp.maximum(beta, jnp.max(jnp.where(m, bidx, -1)))
        sbeta = jnp.maximum(sbeta, jnp.max(jnp.where(m, s_vec, -1)))
        return carry + tot, beta, sbeta
    _, beta, sbeta = lax.fori_loop(
        0, 16, body, (jnp.int32(0), jnp.int32(-1), jnp.int32(-1)))
    return beta, sbeta


def _radix_sort(ai, bk, bv, tk, tv, n, seg, h32a, h32b, lanes, ones, zeros):
    nv = n // 16

    def mmbody(i, car):
        mn, mx = car
        k = ai[pl.ds(bk + i * 16, 16)]
        return jnp.minimum(mn, k), jnp.maximum(mx, k)
    mnv, mxv = lax.fori_loop(0, nv, mmbody, (_i32(2**31 - 1), _i32(0)))
    mink = jnp.min(mnv)
    rng = jnp.max(mxv) - mink

    def subbody(i, _):
        ai[pl.ds(bk + i * 16, 16)] = ai[pl.ds(bk + i * 16, 16)] - mink
        return 0
    lax.fori_loop(0, nv, subbody, 0)

    npasses = jnp.int32(1)
    for j in range(1, 7):
        npasses += (rng >= (1 << (5 * j))).astype(jnp.int32)

    def pass_body(p, _):
        shv = p * 5
        sh = zeros + shv
        for z in range(32):
            ai[pl.ds(h32a + z * 16, 16)] = zeros

        def scan(step, _):
            idx = lanes * seg + step
            k = plsc.load_gather(ai, [_i32(bk) + idx])
            d = lax.shift_right_logical(k, sh) & 31
            plsc.addupdate_scatter(ai, [_i32(h32a) + lanes * 32 + d], ones)
            return 0
        lax.fori_loop(0, seg, scan, 0)

        t0 = zeros
        t1 = zeros
        for l in range(16):
            t0 = t0 + ai[pl.ds(h32a + l * 32, 16)]
            t1 = t1 + ai[pl.ds(h32a + l * 32 + 16, 16)]
        tot0 = jnp.sum(t0)
        tot1 = jnp.sum(t1)
        s0 = tot1 + (tot0 - plsc.cumsum(t0))
        s1 = tot1 - plsc.cumsum(t1)
        acc0 = s0
        acc1 = s1
        for l in range(16):
            ai[pl.ds(h32b + l * 32, 16)] = acc0
            ai[pl.ds(h32b + l * 32 + 16, 16)] = acc1
            acc0 = acc0 + ai[pl.ds(h32a + l * 32, 16)]
            acc1 = acc1 + ai[pl.ds(h32a + l * 32 + 16, 16)]

        def place(step, _):
            idx = lanes * seg + step
            k = plsc.load_gather(ai, [_i32(bk) + idx])
            v = plsc.load_gather(ai, [_i32(bv) + idx])
            d = lax.shift_right_logical(k, sh) & 31
            hidx = _i32(h32b) + lanes * 32 + d
            po = plsc.load_gather(ai, [hidx])
            plsc.store_scatter(ai, [_i32(tk) + po], k)
            plsc.store_scatter(ai, [_i32(tv) + po], v)
            plsc.addupdate_scatter(ai, [hidx], ones)
            return 0
        lax.fori_loop(0, seg, place, 0)

        def copyback(i, _):
            ai[pl.ds(bk + i * 16, 16)] = ai[pl.ds(tk + i * 16, 16)]
            ai[pl.ds(bv + i * 16, 16)] = ai[pl.ds(tv + i * 16, 16)]
            return 0
        lax.fori_loop(0, nv, copyback, 0)
        return 0
    lax.fori_loop(0, npasses, pass_body, 0)
    return mink


def _sc_body(prob_hbm, smh_hbm, smo_hbm, ent_hbm, out_hbm, ai, af, sp):
    cid = lax.axis_index("c")
    sid = lax.axis_index("s")
    tid = sid
    lanes = lax.iota(jnp.int32, L)
    zeros = _i32(0)
    ones = _i32(1)
    ninf = jnp.full((L,), NEG_INF, jnp.float32)

    for z in range(16):
        ai[pl.ds(AI_ZERO + z * 16, 16)] = zeros

    @pl.when(sid == 0)
    def _():
        for lvl in range(4):
            pltpu.sync_copy(
                ai.at[pl.ds(AI_ZERO, 256)],
                sp.at[pl.ds(SP_HIST + cid * 1024 + lvl * 256, 256)])

    plsc.subcore_barrier()

    for mat in range(2):
        src = smh_hbm if mat == 0 else smo_hbm
        ids_base = AI_IDS_SUB if mat == 0 else AI_IDS_OBJ

        def blk_body(blk, _):
            row0 = tid * RPT + blk * 8
            pltpu.sync_copy(src.at[cid, pl.ds(row0 * CPAD, 8 * CPAD)],
                            af.at[pl.ds(AF_ROW0, 8 * CPAD)])

            def row_body(r8, _):
                base = AF_ROW0 + r8 * CPAD

                def chunk(c, car):
                    t1, t2, t3, i1, i2, i3 = car
                    x = af[pl.ds(base + c * 16, 16)]
                    g = c * 16 + lanes
                    b1 = x > t1
                    b2 = x > t2
                    b3 = x > t3
                    nt1 = jnp.where(b1, x, t1)
                    ni1 = jnp.where(b1, g, i1)
                    nt2 = jnp.where(b1, t1, jnp.where(b2, x, t2))
                    ni2 = jnp.where(b1, i1, jnp.where(b2, g, i2))
                    nt3 = jnp.where(b2, t2, jnp.where(b3, x, t3))
                    ni3 = jnp.where(b2, i2, jnp.where(b3, g, i3))
                    return nt1, nt2, nt3, ni1, ni2, ni3
                t1, t2, t3, i1, i2, i3 = lax.fori_loop(
                    0, CPAD // 16, chunk, (ninf, ninf, ninf, zeros, zeros, zeros))
                rl = blk * 8 + r8
                for kx in range(3):
                    v = jnp.max(t1)
                    m = t1 == v
                    g = jnp.min(jnp.where(m, i1, 1 << 30))
                    plsc.store_scatter(
                        ai, [_i32(ids_base) + rl * 3 + kx], zeros + g,
                        mask=lanes == 0)
                    pm = m & (i1 == g)
                    t1 = jnp.where(pm, t2, t1)
                    i1 = jnp.where(pm, i2, i1)
                    t2 = jnp.where(pm, t3, t2)
                    i2 = jnp.where(pm, i3, i2)
                    t3 = jnp.where(pm, ninf, t3)
                return 0
            lax.fori_loop(0, 8, row_body, 0)
            return 0
        lax.fori_loop(0, RPT // 8, blk_body, 0)
        pltpu.sync_copy(
            ai.at[pl.ds(ids_base, RPT * 3)],
            sp.at[pl.ds(SP_IDS + cid * 30720 + mat * 15360 + tid * RPT * 3,
                        RPT * 3)])

    pltpu.sync_copy(prob_hbm.at[cid, pl.ds(tid * SHARD, SHARD)],
                    af.at[pl.ds(AF_PROBV, SHARD)])

    def tbody(step, _):
        x = plsc.load_gather(af, [_i32(AF_PROBV) + lanes * SEG + step])
        ai[pl.ds(AI_KV + step * 16, 16)] = plsc.bitcast(x, jnp.int32)
        return 0
    lax.fori_loop(0, SEG, tbody, 0)

    tstar = jnp.int32(0)
    need = jnp.int32(K1)
    for lvl in range(4):
        sh = 24 - 8 * lvl

        def zh(z, _):
            ai[pl.ds(AI_H2D + z * 16, 16)] = zeros
            return 0
        lax.fori_loop(0, 256, zh, 0)

        def hscan(step, _):
            k = ai[pl.ds(AI_KV + step * 16, 16)]
            d = lax.shift_right_logical(k, _i32(sh)) & 255
            hidx = _i32(AI_H2D) + lanes * 256 + d
            if lvl == 0:
                plsc.addupdate_scatter(ai, [hidx], ones)
            else:
                m = lax.shift_right_logical(k, _i32(sh + 8)) == tstar
                plsc.addupdate_scatter(ai, [hidx], ones, mask=m)
            return 0
        lax.fori_loop(0, SEG, hscan, 0)

        def hmerge(j, _):
            acc = zeros
            for l in range(16):
                acc = acc + ai[pl.ds(AI_H2D + l * 256 + j * 16, 16)]
            ai[pl.ds(AI_HISTM + j * 16, 16)] = acc
            ai[pl.ds(AI_IOTA + j * 16, 16)] = (
                lanes + j * 16 + (SP_HIST + lvl * 256) + cid * 1024)
            return 0
        lax.fori_loop(0, 16, hmerge, 0)
        pltpu.sync_copy(ai.at[pl.ds(AI_HISTM, 256)],
                        sp.at[ai.at[pl.ds(AI_IOTA, 256)]], add=True)
        plsc.subcore_barrier()
        pltpu.sync_copy(sp.at[pl.ds(SP_HIST + cid * 1024 + lvl * 256, 256)],
                        ai.at[pl.ds(AI_HISTM, 256)])
        beta, sbeta = _suffix_find(ai, AI_HISTM, need, lanes)
        need = need - sbeta
        tstar = tstar * 256 + beta
    need_eq = need

    def cbody(step, car):
        cd, ce = car
        k = ai[pl.ds(AI_KV + step * 16, 16)]
        cd = cd + jnp.where(k > tstar, 1, 0)
        ce = ce + jnp.where(k == tstar, 1, 0)
        return cd, ce
    cd_tot, ce_tot = lax.fori_loop(0, SEG, cbody, (zeros, zeros))
    ai[pl.ds(AI_HISTM, 16)] = cd_tot
    ai[pl.ds(AI_HISTM + 16, 16)] = ce_tot
    pltpu.sync_copy(ai.at[pl.ds(AI_HISTM, 16)],
                    sp.at[pl.ds(SP_CNTD + cid * 256 + tid * 16, 16)])
    pltpu.sync_copy(ai.at[pl.ds(AI_HISTM + 16, 16)],
                    sp.at[pl.ds(SP_CNTE + cid * 256 + tid * 16, 16)])
    plsc.subcore_barrier()
    pltpu.sync_copy(sp.at[pl.ds(SP_CNTD + cid * 256, 256)],
                    ai.at[pl.ds(AI_HISTM, 256)])
    pltpu.sync_copy(sp.at[pl.ds(SP_CNTE + cid * 256, 256)],
                    ai.at[pl.ds(AI_IOTA, 256)])
    defbase = zeros
    eqbase = zeros
    n1 = jnp.int32(0)
    netot = jnp.int32(0)
    for j in range(16):
        hd = ai[pl.ds(AI_HISTM + j * 16, 16)]
        he = ai[pl.ds(AI_IOTA + j * 16, 16)]
        exd = n1 + (plsc.cumsum(hd) - hd)
        exe = netot + (plsc.cumsum(he) - he)
        mysel = tid == j
        defbase = jnp.where(mysel, exd, defbase)
        eqbase = jnp.where(mysel, exe, eqbase)
        n1 = n1 + jnp.sum(hd)
        netot = netot + jnp.sum(he)

    spk_base = SP_SURVK + cid * N1P
    trash = _i32(spk_base + N1) + lanes

    def tinit(i, _):
        ai[pl.ds(AI_STGT + i * 16, 16)] = trash
        return 0
    lax.fori_loop(0, SEG, tinit, 0)

    def c2body(step, car):
        cd, ce, cc = car
        k = ai[pl.ds(AI_KV + step * 16, 16)]
        md = k > tstar
        me = k == tstar
        rank_eq = eqbase + ce
        ok_e = me & (rank_eq < need_eq)
        tgt = jnp.where(md, spk_base + defbase + cd,
                        jnp.where(ok_e, spk_base + n1 + rank_eq, trash))
        mst = md | me
        slot = _i32(AI_STGK) + cc * 16 + lanes
        plsc.store_scatter(ai, [slot], k, mask=mst)
        flatidx = tid * SHARD + lanes * SEG + step
        plsc.store_scatter(ai, [slot + (AI_STGV - AI_STGK)], flatidx, mask=mst)
        plsc.store_scatter(ai, [slot + (AI_STGT - AI_STGK)], tgt, mask=mst)
        return (cd + jnp.where(md, 1, 0), ce + jnp.where(me, 1, 0),
                cc + jnp.where(mst, 1, 0))
    _, _, cc_tot = lax.fori_loop(0, SEG, c2body, (zeros, zeros, zeros))
    ngroups = (jnp.max(cc_tot) + 7) // 8

    def gbody(g, _):
        pltpu.sync_copy(ai.at[pl.ds(AI_STGK + g * 128, 128)],
                        sp.at[ai.at[pl.ds(AI_STGT + g * 128, 128)]])
        for v8 in range(8):
            off = AI_STGT + g * 128 + v8 * 16
            ai[pl.ds(off, 16)] = ai[pl.ds(off, 16)] + (SP_SURVI - SP_SURVK)
        pltpu.sync_copy(ai.at[pl.ds(AI_STGV + g * 128, 128)],
                        sp.at[ai.at[pl.ds(AI_STGT + g * 128, 128)]])
        return 0
    lax.fori_loop(0, ngroups, gbody, 0)
    plsc.subcore_barrier()

    @pl.when(sid == 0)
    def _():
        pltpu.sync_copy(sp.at[pl.ds(spk_base, N1P)], ai.at[pl.ds(SO_SK, N1P)])
        pltpu.sync_copy(sp.at[pl.ds(SP_SURVI + cid * N1P, N1P)],
                        ai.at[pl.ds(SO_SV, N1P)])
        ai[pl.ds(SO_SK + K1, 16)] = zeros
        ai[pl.ds(SO_SV + K1, 16)] = zeros
        mink = _radix_sort(ai, SO_SK, SO_SV, SO_KB, SO_VB, N1, SEG2,
                           SO_H32A, SO_H32B, lanes, ones, zeros)

        pltpu.sync_copy(sp.at[pl.ds(SP_IDS + cid * 30720, 15360)],
                        ai.at[pl.ds(SO_SUB, 15360)])
        pltpu.sync_copy(sp.at[pl.ds(SP_IDS + cid * 30720 + 15360, 15360)],
                        ai.at[pl.ds(SO_OBJ, 15360)])
        pltpu.sync_copy(ent_hbm.at[cid], af.at[pl.ds(AF_ENT, ENTP)])

        def trip(ts, _):
            s = ts * 16 + lanes
            r = s // 3
            j = s - r * 3
            flatv = plsc.load_gather(ai, [_i32(SO_SV) + r])
            q = flatv // C
            kk = plsc.load_gather(ai, [_i32(SO_SK) + r])
            pb = plsc.bitcast(kk + mink, jnp.float32)
            sub = plsc.load_gather(ai, [_i32(SO_SUB) + q * 3 + j])
            obj = plsc.load_gather(ai, [_i32(SO_OBJ) + q * 3 + j])
            ssv = plsc.load_gather(af, [_i32(AF_ENT) + sub])
            osv = plsc.load_gather(af, [_i32(AF_ENT) + obj])
            trp = (pb * ssv) * osv
            ok = (s < K1 * 3) & (sub != obj)
            tkey = jnp.where(ok, plsc.bitcast(trp, jnp.int32) + 1, 0)
            ai[pl.ds(SO_TK + ts * 16, 16)] = tkey
            return 0
        lax.fori_loop(0, SEG3, trip, 0)

        t2 = jnp.int32(0)
        need2 = jnp.int32(K2)
        for lvl in range(4):
            sh = 24 - 8 * lvl

            def zh5(z, _):
                ai[pl.ds(SO_H2D5 + z * 16, 16)] = zeros
                return 0
            lax.fori_loop(0, 256, zh5, 0)

            def h5scan(step, _):
                k = plsc.load_gather(ai, [_i32(SO_TK) + lanes * SEG3 + step])
                d = lax.shift_right_logical(k, _i32(sh)) & 255
                hidx = _i32(SO_H2D5) + lanes * 256 + d
                if lvl == 0:
                    plsc.addupdate_scatter(ai, [hidx], ones)
                else:
                    m = lax.shift_right_logical(k, _i32(sh + 8)) == t2
                    plsc.addupdate_scatter(ai, [hidx], ones, mask=m)
                return 0
            lax.fori_loop(0, SEG3, h5scan, 0)

            def h5merge(j, _):
                acc = zeros
                for l in range(16):
                    acc = acc + ai[pl.ds(SO_H2D5 + l * 256 + j * 16, 16)]
                ai[pl.ds(SO_HM5 + j * 16, 16)] = acc
                return 0
            lax.fori_loop(0, 16, h5merge, 0)
            beta, sbeta = _suffix_find(ai, SO_HM5, need2, lanes)
            need2 = need2 - sbeta
            t2 = t2 * 256 + beta

        def c5body(step, car):
            cd, ce = car
            k = plsc.load_gather(ai, [_i32(SO_TK) + lanes * SEG3 + step])
            cd = cd + jnp.where(k > t2, 1, 0)
            ce = ce + jnp.where(k == t2, 1, 0)
            return cd, ce
        cd2, ce2 = lax.fori_loop(0, SEG3, c5body, (zeros, zeros))
        n12 = jnp.sum(cd2)
        defb2 = plsc.cumsum(cd2) - cd2
        eqb2 = plsc.cumsum(ce2) - ce2

        def c5place(step, car):
            cd, ce = car
            pos = lanes * SEG3 + step
            k = plsc.load_gather(ai, [_i32(SO_TK) + pos])
            md = k > t2
            me = k == t2
            rank_eq = eqb2 + ce
            ok_e = me & (rank_eq < need2)
            tgt = jnp.where(md, defb2 + cd, n12 + rank_eq)
            mst = md | ok_e
            plsc.store_scatter(ai, [_i32(SO_K2A) + tgt], k, mask=mst)
            plsc.store_scatter(ai, [_i32(SO_V2A) + tgt], pos, mask=mst)
            return cd + jnp.where(md, 1, 0), ce + jnp.where(me, 1, 0)
        lax.fori_loop(0, SEG3, c5place, (zeros, zeros))

        mink2 = _radix_sort(ai, SO_K2A, SO_V2A, SO_K2B, SO_V2B, K2, SEG4,
                            SO_H32A, SO_H32B, lanes, ones, zeros)
        del mink2

        def obody(os_, _):
            s = ai[pl.ds(SO_V2A + os_ * 16, 16)]
            r = s // 3
            j = s - r * 3
            flatv = plsc.load_gather(ai, [_i32(SO_SV) + r])
            q = flatv // C
            lab = flatv - q * C + 1
            kk = plsc.load_gather(ai, [_i32(SO_SK) + r])
            pb = plsc.bitcast(kk + mink, jnp.float32)
            sub = plsc.load_gather(ai, [_i32(SO_SUB) + q * 3 + j])
            obj = plsc.load_gather(ai, [_i32(SO_OBJ) + q * 3 + j])
            ssv = plsc.load_gather(af, [_i32(AF_ENT) + sub])
            osv = plsc.load_gather(af, [_i32(AF_ENT) + obj])
            trp = (pb * ssv) * osv
            rr6 = (os_ * 16 + lanes) * 6 + AF_OUT
            plsc.store_scatter(af, [rr6], sub.astype(jnp.float32))
            plsc.store_scatter(af, [rr6 + 1], obj.astype(jnp.float32))
            plsc.store_scatter(af, [rr6 + 2], lab.astype(jnp.float32))
            plsc.store_scatter(af, [rr6 + 3], trp)
            plsc.store_scatter(af, [rr6 + 4], pb)
            plsc.store_scatter(af, [rr6 + 5], q.astype(jnp.float32))
            return 0
        lax.fori_loop(0, SEG4, obody, 0)
        pltpu.sync_copy(af.at[pl.ds(AF_OUT, K2 * 6)], out_hbm.at[cid])


@functools.partial(
    pl.kernel,
    out_type=jax.ShapeDtypeStruct((2, K2 * 6), jnp.float32),
    mesh=plsc.VectorSubcoreMesh(core_axis_name="c", subcore_axis_name="s"),
    compiler_params=pltpu.CompilerParams(needs_layout_passes=False),
    scratch_types=[
        pltpu.VMEM((AI_TOTAL,), jnp.int32),
        pltpu.VMEM((AF_TOTAL,), jnp.float32),
        pltpu.VMEM_SHARED((SP_TOTAL,), jnp.int32),
    ],
)
def _sc_kernel(prob_hbm, smh_hbm, smo_hbm, ent_hbm, out_hbm, ai, af, sp):
    _sc_body(prob_hbm, smh_hbm, smo_hbm, ent_hbm, out_hbm, ai, af, sp)


def kernel(pred_rel_logits, pred_hidx, pred_oidx, ent_scores, ent_boxes,
           target_sizes):
    B = pred_rel_logits.shape[0]
    prob = jax.nn.sigmoid(pred_rel_logits).reshape(B, NQ * C)
    probf = jnp.pad(prob, ((0, 0), (0, NFLAT - NQ * C)))
    sm_h = jax.nn.softmax(pred_hidx, axis=-1)
    sm_o = jax.nn.softmax(pred_oidx, axis=-1)
    smh_p = jnp.pad(sm_h, ((0, 0), (0, NROWP - NQ), (0, CPAD - NENT)),
                    constant_values=NEG_INF).reshape(B, NROWP * CPAD)
    smo_p = jnp.pad(sm_o, ((0, 0), (0, NROWP - NQ), (0, CPAD - NENT)),
                    constant_values=NEG_INF).reshape(B, NROWP * CPAD)
    ent_p = jnp.pad(ent_scores, ((0, 0), (0, ENTP - NENT)))
    out = _sc_kernel(probf, smh_p, smo_p, ent_p)
    return out.reshape(B, K2, 6)


# speedup vs baseline: 7.6643x; 1.0053x over previous
import functools

import jax
import jax.numpy as jnp
from jax import lax
from jax.experimental import pallas as pl
from jax.experimental.pallas import tpu as pltpu
from jax.experimental.pallas import tpu_sc as plsc

L = 16
NQ, C, NENT = 5000, 51, 300
NFLAT = 256000
SHARD = NFLAT // 16
SEG = SHARD // L
NROWP = 5120
RPT = NROWP // 16
RPB = 312
CPAD = 304
ENTP = 512
K1 = 5000
N1 = 5008
N1P = 5024
SEG2 = N1 // L
NT = 15008
SEG3 = NT // L
K2 = 2048
SEG4 = K2 // L

AI_IDS_SUB = 0
AI_IDS_OBJ = 960
AI_KV = 1920
AI_H2D = 17920
AI_STGK = 22016
AI_STGV = 38016
AI_STGT = 54016
AI_IOTA = 70016
AI_HISTM = 70272
AI_ZERO = 70528
AI_END = 70784
SO_SK = 0
SO_SV = 5024
SO_KB = 10048
SO_VB = 15072
SO_SUB = 20096
SO_OBJ = 35456
SO_TK = 50816
SO_K2A = 65824
SO_V2A = 67888
SO_K2B = 70784
SO_V2B = 72848
SO_H32A = 74912
SO_H32B = 75424
SO_H2D5 = 75936
SO_HM5 = 80032
AI_TOTAL = 80288

AF_ROW0 = 0
AF_ROW1 = 2432
AF_PROBV = 4864
AF_ENT = 20864
AF_OUT = 21376
AF_TOTAL = 33664

SP_IDS = 0
SP_SURVK = 61440
SP_SURVI = 71488
SP_CNTD = 81536
SP_CNTE = 82048
SP_HIST = 82560
SP_TOTAL = 84608

NEG_INF = float("-inf")


def _i32(v):
    return jnp.full((L,), v, jnp.int32)


def _suffix_find(ai, hm_base, need, lanes):
    def body(t, car):
        carry, beta, sbeta = car
        jj = 15 - t
        h = ai[pl.ds(hm_base + jj * 16, 16)]
        tot = jnp.sum(h)
        cs = plsc.cumsum(h)
        s_vec = carry + (tot - cs)
        m = (s_vec < need) & (s_vec + h >= need)
        bidx = jj * 16 + lanes
        beta = jnp.maximum(beta, jnp.max(jnp.where(m, bidx, -1)))
        sbeta = jnp.maximum(sbeta, jnp.max(jnp.where(m, s_vec, -1)))
        return carry + tot, beta, sbeta
    _, beta, sbeta = lax.fori_loop(
        0, 16, body, (jnp.int32(0), jnp.int32(-1), jnp.int32(-1)))
    return beta, sbeta


def _radix_sort(ai, bk, bv, tk, tv, n, seg, h32a, h32b, lanes, ones, zeros):
    nv = n // 16

    def mmbody(i, car):
        mn, mx = car
        k = ai[pl.ds(bk + i * 16, 16)]
        return jnp.minimum(mn, k), jnp.maximum(mx, k)
    mnv, mxv = lax.fori_loop(0, nv, mmbody, (_i32(2**31 - 1), _i32(0)))
    mink = jnp.min(mnv)
    rng = jnp.max(mxv) - mink

    def subbody(i, _):
        ai[pl.ds(bk + i * 16, 16)] = ai[pl.ds(bk + i * 16, 16)] - mink
        return 0
    lax.fori_loop(0, nv, subbody, 0)

    npasses = jnp.int32(1)
    for j in range(1, 7):
        npasses += (rng >= (1 << (5 * j))).astype(jnp.int32)

    def pass_body(p, _):
        shv = p * 5
        sh = zeros + shv
        for z in range(32):
            ai[pl.ds(h32a + z * 16, 16)] = zeros

        def scan(step, _):
            idx = lanes * seg + step
            k = plsc.load_gather(ai, [_i32(bk) + idx])
            d = lax.shift_right_logical(k, sh) & 31
            plsc.addupdate_scatter(ai, [_i32(h32a) + lanes * 32 + d], ones)
            return 0
        lax.fori_loop(0, seg, scan, 0)

        t0 = zeros
        t1 = zeros
        for l in range(16):
            t0 = t0 + ai[pl.ds(h32a + l * 32, 16)]
            t1 = t1 + ai[pl.ds(h32a + l * 32 + 16, 16)]
        tot0 = jnp.sum(t0)
        tot1 = jnp.sum(t1)
        s0 = tot1 + (tot0 - plsc.cumsum(t0))
        s1 = tot1 - plsc.cumsum(t1)
        acc0 = s0
        acc1 = s1
        for l in range(16):
            ai[pl.ds(h32b + l * 32, 16)] = acc0
            ai[pl.ds(h32b + l * 32 + 16, 16)] = acc1
            acc0 = acc0 + ai[pl.ds(h32a + l * 32, 16)]
            acc1 = acc1 + ai[pl.ds(h32a + l * 32 + 16, 16)]

        def place(step, _):
            idx = lanes * seg + step
            k = plsc.load_gather(ai, [_i32(bk) + idx])
            v = plsc.load_gather(ai, [_i32(bv) + idx])
            d = lax.shift_right_logical(k, sh) & 31
            hidx = _i32(h32b) + lanes * 32 + d
            po = plsc.load_gather(ai, [hidx])
            plsc.store_scatter(ai, [_i32(tk) + po], k)
            plsc.store_scatter(ai, [_i32(tv) + po], v)
            plsc.addupdate_scatter(ai, [hidx], ones)
            return 0
        lax.fori_loop(0, seg, place, 0)

        def copyback(i, _):
            ai[pl.ds(bk + i * 16, 16)] = ai[pl.ds(tk + i * 16, 16)]
            ai[pl.ds(bv + i * 16, 16)] = ai[pl.ds(tv + i * 16, 16)]
            return 0
        lax.fori_loop(0, nv, copyback, 0)
        return 0
    lax.fori_loop(0, npasses, pass_body, 0)
    return mink


def _sc_body(prob_hbm, smh_hbm, smo_hbm, tailh_hbm, tailo_hbm, ent_hbm,
             out_hbm, ai, af, sp):
    cid = lax.axis_index("c")
    sid = lax.axis_index("s")
    tid = sid
    lanes = lax.iota(jnp.int32, L)
    zeros = _i32(0)
    ones = _i32(1)
    ninf = jnp.full((L,), NEG_INF, jnp.float32)

    for z in range(16):
        ai[pl.ds(AI_ZERO + z * 16, 16)] = zeros

    @pl.when(sid == 0)
    def _():
        for lvl in range(4):
            pltpu.sync_copy(
                ai.at[pl.ds(AI_ZERO, 256)],
                sp.at[pl.ds(SP_HIST + cid * 1024 + lvl * 256, 256)])

    plsc.subcore_barrier()

    for mat in range(2):
        src = smh_hbm if mat == 0 else smo_hbm
        ids_base = AI_IDS_SUB if mat == 0 else AI_IDS_OBJ

        def insert(x, g, car):
            t1, t2, t3, i1, i2, i3 = car
            b1 = x > t1
            b2 = x > t2
            b3 = x > t3
            nt1 = jnp.where(b1, x, t1)
            ni1 = jnp.where(b1, g, i1)
            nt2 = jnp.where(b1, t1, jnp.where(b2, x, t2))
            ni2 = jnp.where(b1, i1, jnp.where(b2, g, i2))
            nt3 = jnp.where(b2, t2, jnp.where(b3, x, t3))
            ni3 = jnp.where(b2, i2, jnp.where(b3, g, i3))
            return nt1, nt2, nt3, ni1, ni2, ni3

        def do_rows(delta, rl0):
            def row_body(r8, _):
                base = AF_ROW0 + delta + r8 * NENT

                def chunk(c, car):
                    x = plsc.load_gather(af, [_i32(base) + c * 16 + lanes])
                    return insert(x, c * 16 + lanes, car)
                car = lax.fori_loop(
                    0, NENT // 16, chunk,
                    (ninf, ninf, ninf, zeros, zeros, zeros))
                gt = (NENT // 16) * 16 + lanes
                xt = plsc.load_gather(af, [_i32(base) + gt])
                xt = jnp.where(gt < NENT, xt, ninf)
                t1, t2, t3, i1, i2, i3 = insert(xt, gt, car)
                rl = rl0 + r8
                for kx in range(3):
                    v = jnp.max(t1)
                    m = t1 == v
                    g = jnp.min(jnp.where(m, i1, 1 << 30))
                    plsc.store_scatter(
                        ai, [_i32(ids_base) + rl * 3 + kx], zeros + g,
                        mask=lanes == 0)
                    pm = m & (i1 == g)
                    t1 = jnp.where(pm, t2, t1)
                    i1 = jnp.where(pm, i2, i1)
                    t2 = jnp.where(pm, t3, t2)
                    i2 = jnp.where(pm, i3, i2)
                    t3 = jnp.where(pm, ninf, t3)
                return 0
            lax.fori_loop(0, 8, row_body, 0)

        def blk_body(blk, _):
            off = (tid * RPB + blk * 8) * NENT
            astart = pl.multiple_of((off // 128) * 128, 128)
            pltpu.sync_copy(src.at[cid, pl.ds(astart, 2560)],
                            af.at[pl.ds(AF_ROW0, 2560)])
            do_rows(off - astart, blk * 8)
            return 0
        lax.fori_loop(0, RPB // 8, blk_body, 0)

        @pl.when(tid == 15)
        def _():
            tail = tailh_hbm if mat == 0 else tailo_hbm
            pltpu.sync_copy(tail.at[cid], af.at[pl.ds(AF_ROW0, 2560)])
            do_rows(0, RPB)
        ids_sp = SP_IDS + cid * 30720 + mat * 15360 + tid * RPB * 3

        @pl.when(tid < 15)
        def _():
            pltpu.sync_copy(ai.at[pl.ds(ids_base, RPB * 3)],
                            sp.at[pl.ds(ids_sp, RPB * 3)])

        @pl.when(tid == 15)
        def _():
            pltpu.sync_copy(ai.at[pl.ds(ids_base, (RPB + 8) * 3)],
                            sp.at[pl.ds(ids_sp, (RPB + 8) * 3)])

    pltpu.sync_copy(prob_hbm.at[cid, pl.ds(tid * SHARD, SHARD)],
                    af.at[pl.ds(AF_PROBV, SHARD)])

    def tbody(step, _):
        x = plsc.load_gather(af, [_i32(AF_PROBV) + lanes * SEG + step])
        ai[pl.ds(AI_KV + step * 16, 16)] = plsc.bitcast(x, jnp.int32)
        return 0
    lax.fori_loop(0, SEG, tbody, 0)

    tstar = jnp.int32(0)
    need = jnp.int32(K1)
    for lvl in range(4):
        sh = 24 - 8 * lvl

        def zh(z, _):
            ai[pl.ds(AI_H2D + z * 16, 16)] = zeros
            return 0
        lax.fori_loop(0, 256, zh, 0)

        def hscan(step, _):
            k = ai[pl.ds(AI_KV + step * 16, 16)]
            d = lax.shift_right_logical(k, _i32(sh)) & 255
            hidx = _i32(AI_H2D) + lanes * 256 + d
            if lvl == 0:
                plsc.addupdate_scatter(ai, [hidx], ones)
            else:
                m = lax.shift_right_logical(k, _i32(sh + 8)) == tstar
                plsc.addupdate_scatter(ai, [hidx], ones, mask=m)
            return 0
        lax.fori_loop(0, SEG, hscan, 0)

        def hmerge(j, _):
            acc = zeros
            for l in range(16):
                acc = acc + ai[pl.ds(AI_H2D + l * 256 + j * 16, 16)]
            ai[pl.ds(AI_HISTM + j * 16, 16)] = acc
            ai[pl.ds(AI_IOTA + j * 16, 16)] = (
                lanes + j * 16 + (SP_HIST + lvl * 256) + cid * 1024)
            return 0
        lax.fori_loop(0, 16, hmerge, 0)
        pltpu.sync_copy(ai.at[pl.ds(AI_HISTM, 256)],
                        sp.at[ai.at[pl.ds(AI_IOTA, 256)]], add=True)
        plsc.subcore_barrier()
        pltpu.sync_copy(sp.at[pl.ds(SP_HIST + cid * 1024 + lvl * 256, 256)],
                        ai.at[pl.ds(AI_HISTM, 256)])
        beta, sbeta = _suffix_find(ai, AI_HISTM, need, lanes)
        need = need - sbeta
        tstar = tstar * 256 + beta
    need_eq = need

    def cbody(step, car):
        cd, ce = car
        k = ai[pl.ds(AI_KV + step * 16, 16)]
        cd = cd + jnp.where(k > tstar, 1, 0)
        ce = ce + jnp.where(k == tstar, 1, 0)
        return cd, ce
    cd_tot, ce_tot = lax.fori_loop(0, SEG, cbody, (zeros, zeros))
    ai[pl.ds(AI_HISTM, 16)] = cd_tot
    ai[pl.ds(AI_HISTM + 16, 16)] = ce_tot
    pltpu.sync_copy(ai.at[pl.ds(AI_HISTM, 16)],
                    sp.at[pl.ds(SP_CNTD + cid * 256 + tid * 16, 16)])
    pltpu.sync_copy(ai.at[pl.ds(AI_HISTM + 16, 16)],
                    sp.at[pl.ds(SP_CNTE + cid * 256 + tid * 16, 16)])
    plsc.subcore_barrier()
    pltpu.sync_copy(sp.at[pl.ds(SP_CNTD + cid * 256, 256)],
                    ai.at[pl.ds(AI_HISTM, 256)])
    pltpu.sync_copy(sp.at[pl.ds(SP_CNTE + cid * 256, 256)],
                    ai.at[pl.ds(AI_IOTA, 256)])
    defbase = zeros
    eqbase = zeros
    n1 = jnp.int32(0)
    netot = jnp.int32(0)
    for j in range(16):
        hd = ai[pl.ds(AI_HISTM + j * 16, 16)]
        he = ai[pl.ds(AI_IOTA + j * 16, 16)]
        exd = n1 + (plsc.cumsum(hd) - hd)
        exe = netot + (plsc.cumsum(he) - he)
        mysel = tid == j
        defbase = jnp.where(mysel, exd, defbase)
        eqbase = jnp.where(mysel, exe, eqbase)
        n1 = n1 + jnp.sum(hd)
        netot = netot + jnp.sum(he)

    spk_base = SP_SURVK + cid * N1P
    trash = _i32(spk_base + N1) + lanes

    def tinit(i, _):
        ai[pl.ds(AI_STGT + i * 16, 16)] = trash
        return 0
    lax.fori_loop(0, SEG, tinit, 0)

    def c2body(step, car):
        cd, ce, cc = car
        k = ai[pl.ds(AI_KV + step * 16, 16)]
        md = k > tstar
        me = k == tstar
        rank_eq = eqbase + ce
        ok_e = me & (rank_eq < need_eq)
        tgt = jnp.where(md, spk_base + defbase + cd,
                        jnp.where(ok_e, spk_base + n1 + rank_eq, trash))
        mst = md | me
        slot = _i32(AI_STGK) + cc * 16 + lanes
        plsc.store_scatter(ai, [slot], k, mask=mst)
        flatidx = tid * SHARD + lanes * SEG + step
        plsc.store_scatter(ai, [slot + (AI_STGV - AI_STGK)], flatidx, mask=mst)
        plsc.store_scatter(ai, [slot + (AI_STGT - AI_STGK)], tgt, mask=mst)
        return (cd + jnp.where(md, 1, 0), ce + jnp.where(me, 1, 0),
                cc + jnp.where(mst, 1, 0))
    _, _, cc_tot = lax.fori_loop(0, SEG, c2body, (zeros, zeros, zeros))
    ngroups = (jnp.max(cc_tot) + 7) // 8

    def gbody(g, _):
        pltpu.sync_copy(ai.at[pl.ds(AI_STGK + g * 128, 128)],
                        sp.at[ai.at[pl.ds(AI_STGT + g * 128, 128)]])
        for v8 in range(8):
            off = AI_STGT + g * 128 + v8 * 16
            ai[pl.ds(off, 16)] = ai[pl.ds(off, 16)] + (SP_SURVI - SP_SURVK)
        pltpu.sync_copy(ai.at[pl.ds(AI_STGV + g * 128, 128)],
                        sp.at[ai.at[pl.ds(AI_STGT + g * 128, 128)]])
        return 0
    lax.fori_loop(0, ngroups, gbody, 0)
    plsc.subcore_barrier()

    @pl.when(sid == 0)
    def _():
        pltpu.sync_copy(sp.at[pl.ds(spk_base, N1P)], ai.at[pl.ds(SO_SK, N1P)])
        pltpu.sync_copy(sp.at[pl.ds(SP_SURVI + cid * N1P, N1P)],
                        ai.at[pl.ds(SO_SV, N1P)])
        ai[pl.ds(SO_SK + K1, 16)] = zeros
        ai[pl.ds(SO_SV + K1, 16)] = zeros
        mink = _radix_sort(ai, SO_SK, SO_SV, SO_KB, SO_VB, N1, SEG2,
                           SO_H32A, SO_H32B, lanes, ones, zeros)

        pltpu.sync_copy(sp.at[pl.ds(SP_IDS + cid * 30720, 15360)],
                        ai.at[pl.ds(SO_SUB, 15360)])
        pltpu.sync_copy(sp.at[pl.ds(SP_IDS + cid * 30720 + 15360, 15360)],
                        ai.at[pl.ds(SO_OBJ, 15360)])
        pltpu.sync_copy(ent_hbm.at[cid], af.at[pl.ds(AF_ENT, ENTP)])

        def trip(ts, _):
            s = ts * 16 + lanes
            r = s // 3
            j = s - r * 3
            flatv = plsc.load_gather(ai, [_i32(SO_SV) + r])
            q = flatv // C
            kk = plsc.load_gather(ai, [_i32(SO_SK) + r])
            pb = plsc.bitcast(kk + mink, jnp.float32)
            sub = plsc.load_gather(ai, [_i32(SO_SUB) + q * 3 + j])
            obj = plsc.load_gather(ai, [_i32(SO_OBJ) + q * 3 + j])
            ssv = plsc.load_gather(af, [_i32(AF_ENT) + sub])
            osv = plsc.load_gather(af, [_i32(AF_ENT) + obj])
            trp = (pb * ssv) * osv
            ok = (s < K1 * 3) & (sub != obj)
            tkey = jnp.where(ok, plsc.bitcast(trp, jnp.int32) + 1, 0)
            ai[pl.ds(SO_TK + ts * 16, 16)] = tkey
            return 0
        lax.fori_loop(0, SEG3, trip, 0)

        t2 = jnp.int32(0)
        need2 = jnp.int32(K2)
        for lvl in range(4):
            sh = 24 - 8 * lvl

            def zh5(z, _):
                ai[pl.ds(SO_H2D5 + z * 16, 16)] = zeros
                return 0
            lax.fori_loop(0, 256, zh5, 0)

            def h5scan(step, _):
                k = plsc.load_gather(ai, [_i32(SO_TK) + lanes * SEG3 + step])
                d = lax.shift_right_logical(k, _i32(sh)) & 255
                hidx = _i32(SO_H2D5) + lanes * 256 + d
                if lvl == 0:
                    plsc.addupdate_scatter(ai, [hidx], ones)
                else:
                    m = lax.shift_right_logical(k, _i32(sh + 8)) == t2
                    plsc.addupdate_scatter(ai, [hidx], ones, mask=m)
                return 0
            lax.fori_loop(0, SEG3, h5scan, 0)

            def h5merge(j, _):
                acc = zeros
                for l in range(16):
                    acc = acc + ai[pl.ds(SO_H2D5 + l * 256 + j * 16, 16)]
                ai[pl.ds(SO_HM5 + j * 16, 16)] = acc
                return 0
            lax.fori_loop(0, 16, h5merge, 0)
            beta, sbeta = _suffix_find(ai, SO_HM5, need2, lanes)
            need2 = need2 - sbeta
            t2 = t2 * 256 + beta

        def c5body(step, car):
            cd, ce = car
            k = plsc.load_gather(ai, [_i32(SO_TK) + lanes * SEG3 + step])
            cd = cd + jnp.where(k > t2, 1, 0)
            ce = ce + jnp.where(k == t2, 1, 0)
            return cd, ce
        cd2, ce2 = lax.fori_loop(0, SEG3, c5body, (zeros, zeros))
        n12 = jnp.sum(cd2)
        defb2 = plsc.cumsum(cd2) - cd2
        eqb2 = plsc.cumsum(ce2) - ce2

        def c5place(step, car):
            cd, ce = car
            pos = lanes * SEG3 + step
            k = plsc.load_gather(ai, [_i32(SO_TK) + pos])
            md = k > t2
            me = k == t2
            rank_eq = eqb2 + ce
            ok_e = me & (rank_eq < need2)
            tgt = jnp.where(md, defb2 + cd, n12 + rank_eq)
            mst = md | ok_e
            plsc.store_scatter(ai, [_i32(SO_K2A) + tgt], k, mask=mst)
            plsc.store_scatter(ai, [_i32(SO_V2A) + tgt], pos, mask=mst)
            return cd + jnp.where(md, 1, 0), ce + jnp.where(me, 1, 0)
        lax.fori_loop(0, SEG3, c5place, (zeros, zeros))

        mink2 = _radix_sort(ai, SO_K2A, SO_V2A, SO_K2B, SO_V2B, K2, SEG4,
                            SO_H32A, SO_H32B, lanes, ones, zeros)
        del mink2

        def obody(os_, _):
            s = ai[pl.ds(SO_V2A + os_ * 16, 16)]
            r = s // 3
            j = s - r * 3
            flatv = plsc.load_gather(ai, [_i32(SO_SV) + r])
            q = flatv // C
            lab = flatv - q * C + 1
            kk = plsc.load_gather(ai, [_i32(SO_SK) + r])
            pb = plsc.bitcast(kk + mink, jnp.float32)
            sub = plsc.load_gather(ai, [_i32(SO_SUB) + q * 3 + j])
            obj = plsc.load_gather(ai, [_i32(SO_OBJ) + q * 3 + j])
            ssv = plsc.load_gather(af, [_i32(AF_ENT) + sub])
            osv = plsc.load_gather(af, [_i32(AF_ENT) + obj])
            trp = (pb * ssv) * osv
            rr6 = (os_ * 16 + lanes) * 6 + AF_OUT
            plsc.store_scatter(af, [rr6], sub.astype(jnp.float32))
            plsc.store_scatter(af, [rr6 + 1], obj.astype(jnp.float32))
            plsc.store_scatter(af, [rr6 + 2], lab.astype(jnp.float32))
            plsc.store_scatter(af, [rr6 + 3], trp)
            plsc.store_scatter(af, [rr6 + 4], pb)
            plsc.store_scatter(af, [rr6 + 5], q.astype(jnp.float32))
            return 0
        lax.fori_loop(0, SEG4, obody, 0)
        pltpu.sync_copy(af.at[pl.ds(AF_OUT, K2 * 6)], out_hbm.at[cid])


@functools.partial(
    pl.kernel,
    out_type=jax.ShapeDtypeStruct((2, K2 * 6), jnp.float32),
    mesh=plsc.VectorSubcoreMesh(core_axis_name="c", subcore_axis_name="s"),
    compiler_params=pltpu.CompilerParams(needs_layout_passes=False),
    scratch_types=[
        pltpu.VMEM((AI_TOTAL,), jnp.int32),
        pltpu.VMEM((AF_TOTAL,), jnp.float32),
        pltpu.VMEM_SHARED((SP_TOTAL,), jnp.int32),
    ],
)
def _sc_kernel(prob_hbm, smh_hbm, smo_hbm, tailh_hbm, tailo_hbm, ent_hbm,
               out_hbm, ai, af, sp):
    _sc_body(prob_hbm, smh_hbm, smo_hbm, tailh_hbm, tailo_hbm, ent_hbm,
             out_hbm, ai, af, sp)


def kernel(pred_rel_logits, pred_hidx, pred_oidx, ent_scores, ent_boxes,
           target_sizes):
    B = pred_rel_logits.shape[0]
    prob = jax.nn.sigmoid(pred_rel_logits).reshape(B, NQ * C)
    probf = jnp.pad(prob, ((0, 0), (0, NFLAT - NQ * C)))
    smh_p = jax.nn.softmax(pred_hidx, axis=-1).reshape(B, NQ * NENT)
    smo_p = jax.nn.softmax(pred_oidx, axis=-1).reshape(B, NQ * NENT)
    ent_p = jnp.pad(ent_scores, ((0, 0), (0, ENTP - NENT)))
    tail_h = jnp.pad(pred_hidx[:, RPB * 16:, :].reshape(B, 8 * NENT),
                     ((0, 0), (0, 2560 - 8 * NENT)))
    tail_o = jnp.pad(pred_oidx[:, RPB * 16:, :].reshape(B, 8 * NENT),
                     ((0, 0), (0, 2560 - 8 * NENT)))
    out = _sc_kernel(probf, smh_p, smo_p, tail_h, tail_o, ent_p)
    return out.reshape(B, K2, 6)


# speedup vs baseline: 7.6647x; 1.0001x over previous
import functools

import jax
import jax.numpy as jnp
from jax import lax
from jax.experimental import pallas as pl
from jax.experimental.pallas import tpu as pltpu
from jax.experimental.pallas import tpu_sc as plsc

L = 16
NQ, C, NENT = 5000, 51, 300
NFLAT = 256000
SHARD = NFLAT // 16
SEG = SHARD // L
NROWP = 5120
RPT = NROWP // 16
RPB = 312
CPAD = 304
ENTP = 512
K1 = 5000
N1 = 5008
N1P = 5024
SEG2 = N1 // L
NT = 15008
SEG3 = NT // L
K2 = 2048
SEG4 = K2 // L

AI_IDS_SUB = 0
AI_IDS_OBJ = 960
AI_KV = 1920
AI_H2D = 17920
AI_STGK = 22016
AI_STGV = 38016
AI_STGT = 54016
AI_IOTA = 70016
AI_HISTM = 70272
AI_ZERO = 70528
AI_END = 70784
SO_SK = 0
SO_SV = 5024
SO_KB = 10048
SO_VB = 15072
SO_SUB = 20096
SO_OBJ = 35456
SO_TK = 50816
SO_K2A = 65824
SO_V2A = 67888
SO_K2B = 70784
SO_V2B = 72848
SO_H32A = 74912
SO_H32B = 75424
SO_H2D5 = 75936
SO_HM5 = 80032
AI_TOTAL = 80288

AF_ROW0 = 0
AF_ROW1 = 2432
AF_PROBV = 4864
AF_ENT = 20864
AF_OUT = 21376
AF_TOTAL = 33664

SP_IDS = 0
SP_SURVK = 61440
SP_SURVI = 71488
SP_CNTD = 81536
SP_CNTE = 82048
SP_HIST = 82560
SP_TOTAL = 84608

NEG_INF = float("-inf")


def _i32(v):
    return jnp.full((L,), v, jnp.int32)


def _suffix_find(ai, hm_base, need, lanes):
    def body(t, car):
        carry, beta, sbeta = car
        jj = 15 - t
        h = ai[pl.ds(hm_base + jj * 16, 16)]
        tot = jnp.sum(h)
        cs = plsc.cumsum(h)
        s_vec = carry + (tot - cs)
        m = (s_vec < need) & (s_vec + h >= need)
        bidx = jj * 16 + lanes
        beta = jnp.maximum(beta, jnp.max(jnp.where(m, bidx, -1)))
        sbeta = jnp.maximum(sbeta, jnp.max(jnp.where(m, s_vec, -1)))
        return carry + tot, beta, sbeta
    _, beta, sbeta = lax.fori_loop(
        0, 16, body, (jnp.int32(0), jnp.int32(-1), jnp.int32(-1)))
    return beta, sbeta


def _radix_sort(ai, bk, bv, tk, tv, n, seg, h32a, h32b, lanes, ones, zeros):
    nv = n // 16

    def mmbody(i, car):
        mn, mx = car
        k = ai[pl.ds(bk + i * 16, 16)]
        return jnp.minimum(mn, k), jnp.maximum(mx, k)
    mnv, mxv = lax.fori_loop(0, nv, mmbody, (_i32(2**31 - 1), _i32(0)))
    mink = jnp.min(mnv)
    rng = jnp.max(mxv) - mink

    def subbody(i, _):
        ai[pl.ds(bk + i * 16, 16)] = ai[pl.ds(bk + i * 16, 16)] - mink
        return 0
    lax.fori_loop(0, nv, subbody, 0)

    npasses = jnp.int32(1)
    for j in range(1, 7):
        npasses += (rng >= (1 << (5 * j))).astype(jnp.int32)

    def pass_body(p, _):
        shv = p * 5
        sh = zeros + shv
        for z in range(32):
            ai[pl.ds(h32a + z * 16, 16)] = zeros

        def scan(step, _):
            idx = lanes * seg + step
            k = plsc.load_gather(ai, [_i32(bk) + idx])
            d = lax.shift_right_logical(k, sh) & 31
            plsc.addupdate_scatter(ai, [_i32(h32a) + lanes * 32 + d], ones)
            return 0
        lax.fori_loop(0, seg, scan, 0)

        t0 = zeros
        t1 = zeros
        for l in range(16):
            t0 = t0 + ai[pl.ds(h32a + l * 32, 16)]
            t1 = t1 + ai[pl.ds(h32a + l * 32 + 16, 16)]
        tot0 = jnp.sum(t0)
        tot1 = jnp.sum(t1)
        s0 = tot1 + (tot0 - plsc.cumsum(t0))
        s1 = tot1 - plsc.cumsum(t1)
        acc0 = s0
        acc1 = s1
        for l in range(16):
            ai[pl.ds(h32b + l * 32, 16)] = acc0
            ai[pl.ds(h32b + l * 32 + 16, 16)] = acc1
            acc0 = acc0 + ai[pl.ds(h32a + l * 32, 16)]
            acc1 = acc1 + ai[pl.ds(h32a + l * 32 + 16, 16)]

        def place(step, _):
            idx = lanes * seg + step
            k = plsc.load_gather(ai, [_i32(bk) + idx])
            v = plsc.load_gather(ai, [_i32(bv) + idx])
            d = lax.shift_right_logical(k, sh) & 31
            hidx = _i32(h32b) + lanes * 32 + d
            po = plsc.load_gather(ai, [hidx])
            plsc.store_scatter(ai, [_i32(tk) + po], k)
            plsc.store_scatter(ai, [_i32(tv) + po], v)
            plsc.addupdate_scatter(ai, [hidx], ones)
            return 0
        lax.fori_loop(0, seg, place, 0)

        def copyback(i, _):
            ai[pl.ds(bk + i * 16, 16)] = ai[pl.ds(tk + i * 16, 16)]
            ai[pl.ds(bv + i * 16, 16)] = ai[pl.ds(tv + i * 16, 16)]
            return 0
        lax.fori_loop(0, nv, copyback, 0)
        return 0
    lax.fori_loop(0, npasses, pass_body, 0)
    return mink


def _sc_body(prob_hbm, smh_hbm, smo_hbm, tailh_hbm, tailo_hbm, ent_hbm,
             out_hbm, ai, af, sp):
    cid = lax.axis_index("c")
    sid = lax.axis_index("s")
    tid = sid
    lanes = lax.iota(jnp.int32, L)
    zeros = _i32(0)
    ones = _i32(1)
    ninf = jnp.full((L,), NEG_INF, jnp.float32)

    for z in range(16):
        ai[pl.ds(AI_ZERO + z * 16, 16)] = zeros

    @pl.when(sid == 0)
    def _():
        for lvl in range(4):
            pltpu.sync_copy(
                ai.at[pl.ds(AI_ZERO, 256)],
                sp.at[pl.ds(SP_HIST + cid * 1024 + lvl * 256, 256)])

    plsc.subcore_barrier()

    for mat in range(2):
        src = smh_hbm if mat == 0 else smo_hbm
        ids_base = AI_IDS_SUB if mat == 0 else AI_IDS_OBJ

        def insert(x, g, car):
            t1, t2, t3, i1, i2, i3 = car
            b1 = x > t1
            b2 = x > t2
            b3 = x > t3
            nt1 = jnp.where(b1, x, t1)
            ni1 = jnp.where(b1, g, i1)
            nt2 = jnp.where(b1, t1, jnp.where(b2, x, t2))
            ni2 = jnp.where(b1, i1, jnp.where(b2, g, i2))
            nt3 = jnp.where(b2, t2, jnp.where(b3, x, t3))
            ni3 = jnp.where(b2, i2, jnp.where(b3, g, i3))
            return nt1, nt2, nt3, ni1, ni2, ni3

        def do_rows(delta, rl0):
            def row_body(r8, _):
                base = AF_ROW0 + delta + r8 * NENT

                def chunk(c, car):
                    x = plsc.load_gather(af, [_i32(base) + c * 16 + lanes])
                    return insert(x, c * 16 + lanes, car)
                car = lax.fori_loop(
                    0, NENT // 16, chunk,
                    (ninf, ninf, ninf, zeros, zeros, zeros))
                gt = (NENT // 16) * 16 + lanes
                xt = plsc.load_gather(af, [_i32(base) + gt])
                xt = jnp.where(gt < NENT, xt, ninf)
                t1, t2, t3, i1, i2, i3 = insert(xt, gt, car)
                rl = rl0 + r8
                for kx in range(3):
                    v = jnp.max(t1)
                    m = t1 == v
                    g = jnp.min(jnp.where(m, i1, 1 << 30))
                    plsc.store_scatter(
                        ai, [_i32(ids_base) + rl * 3 + kx], zeros + g,
                        mask=lanes == 0)
                    pm = m & (i1 == g)
                    t1 = jnp.where(pm, t2, t1)
                    i1 = jnp.where(pm, i2, i1)
                    t2 = jnp.where(pm, t3, t2)
                    i2 = jnp.where(pm, i3, i2)
                    t3 = jnp.where(pm, ninf, t3)
                return 0
            lax.fori_loop(0, 8, row_body, 0)

        def blk_body(blk, _):
            off = (tid * RPB + blk * 8) * NENT
            astart = pl.multiple_of((off // 128) * 128, 128)
            pltpu.sync_copy(src.at[cid, pl.ds(astart, 2560)],
                            af.at[pl.ds(AF_ROW0, 2560)])
            do_rows(off - astart, blk * 8)
            return 0
        lax.fori_loop(0, RPB // 8, blk_body, 0)

        @pl.when(tid == 15)
        def _():
            tail = tailh_hbm if mat == 0 else tailo_hbm
            pltpu.sync_copy(tail.at[cid], af.at[pl.ds(AF_ROW0, 2560)])
            do_rows(0, RPB)
        ids_sp = SP_IDS + cid * 30720 + mat * 15360 + tid * RPB * 3

        @pl.when(tid < 15)
        def _():
            pltpu.sync_copy(ai.at[pl.ds(ids_base, RPB * 3)],
                            sp.at[pl.ds(ids_sp, RPB * 3)])

        @pl.when(tid == 15)
        def _():
            pltpu.sync_copy(ai.at[pl.ds(ids_base, (RPB + 8) * 3)],
                            sp.at[pl.ds(ids_sp, (RPB + 8) * 3)])

    pltpu.sync_copy(prob_hbm.at[cid, pl.ds(tid * SHARD, SHARD)],
                    af.at[pl.ds(AF_PROBV, SHARD)])

    def tbody(step, _):
        x = plsc.load_gather(af, [_i32(AF_PROBV) + lanes * SEG + step])
        ai[pl.ds(AI_KV + step * 16, 16)] = plsc.bitcast(x, jnp.int32)
        return 0
    lax.fori_loop(0, SEG, tbody, 0)

    tstar = jnp.int32(0)
    need = jnp.int32(K1)
    for lvl in range(4):
        sh = 24 - 8 * lvl

        def zh(z, _):
            ai[pl.ds(AI_H2D + z * 16, 16)] = zeros
            return 0
        lax.fori_loop(0, 256, zh, 0)

        def hscan(step, _):
            k = ai[pl.ds(AI_KV + step * 16, 16)]
            d = lax.shift_right_logical(k, _i32(sh)) & 255
            hidx = _i32(AI_H2D) + lanes * 256 + d
            if lvl == 0:
                plsc.addupdate_scatter(ai, [hidx], ones)
            else:
                m = lax.shift_right_logical(k, _i32(sh + 8)) == tstar
                plsc.addupdate_scatter(ai, [hidx], ones, mask=m)
            return 0
        lax.fori_loop(0, SEG, hscan, 0)

        def hmerge(j, _):
            acc = zeros
            for l in range(16):
                acc = acc + ai[pl.ds(AI_H2D + l * 256 + j * 16, 16)]
            ai[pl.ds(AI_HISTM + j * 16, 16)] = acc
            ai[pl.ds(AI_IOTA + j * 16, 16)] = (
                lanes + j * 16 + (SP_HIST + lvl * 256) + cid * 1024)
            return 0
        lax.fori_loop(0, 16, hmerge, 0)
        pltpu.sync_copy(ai.at[pl.ds(AI_HISTM, 256)],
                        sp.at[ai.at[pl.ds(AI_IOTA, 256)]], add=True)
        plsc.subcore_barrier()
        pltpu.sync_copy(sp.at[pl.ds(SP_HIST + cid * 1024 + lvl * 256, 256)],
                        ai.at[pl.ds(AI_HISTM, 256)])
        beta, sbeta = _suffix_find(ai, AI_HISTM, need, lanes)
        need = need - sbeta
        tstar = tstar * 256 + beta
    need_eq = need

    def cbody(step, car):
        cd, ce = car
        k = ai[pl.ds(AI_KV + step * 16, 16)]
        cd = cd + jnp.where(k > tstar, 1, 0)
        ce = ce + jnp.where(k == tstar, 1, 0)
        return cd, ce
    cd_tot, ce_tot = lax.fori_loop(0, SEG, cbody, (zeros, zeros))
    ai[pl.ds(AI_HISTM, 16)] = cd_tot
    ai[pl.ds(AI_HISTM + 16, 16)] = ce_tot
    pltpu.sync_copy(ai.at[pl.ds(AI_HISTM, 16)],
                    sp.at[pl.ds(SP_CNTD + cid * 256 + tid * 16, 16)])
    pltpu.sync_copy(ai.at[pl.ds(AI_HISTM + 16, 16)],
                    sp.at[pl.ds(SP_CNTE + cid * 256 + tid * 16, 16)])
    plsc.subcore_barrier()
    pltpu.sync_copy(sp.at[pl.ds(SP_CNTD + cid * 256, 256)],
                    ai.at[pl.ds(AI_HISTM, 256)])
    pltpu.sync_copy(sp.at[pl.ds(SP_CNTE + cid * 256, 256)],
                    ai.at[pl.ds(AI_IOTA, 256)])
    defbase = zeros
    eqbase = zeros
    n1 = jnp.int32(0)
    netot = jnp.int32(0)
    for j in range(16):
        hd = ai[pl.ds(AI_HISTM + j * 16, 16)]
        he = ai[pl.ds(AI_IOTA + j * 16, 16)]
        exd = n1 + (plsc.cumsum(hd) - hd)
        exe = netot + (plsc.cumsum(he) - he)
        mysel = tid == j
        defbase = jnp.where(mysel, exd, defbase)
        eqbase = jnp.where(mysel, exe, eqbase)
        n1 = n1 + jnp.sum(hd)
        netot = netot + jnp.sum(he)

    spk_base = SP_SURVK + cid * N1P
    trash = _i32(spk_base + N1) + lanes

    def tinit(i, _):
        ai[pl.ds(AI_STGT + i * 16, 16)] = trash
        return 0
    lax.fori_loop(0, SEG, tinit, 0)

    def c2body(step, car):
        cd, ce, cc = car
        k = ai[pl.ds(AI_KV + step * 16, 16)]
        md = k > tstar
        me = k == tstar
        rank_eq = eqbase + ce
        ok_e = me & (rank_eq < need_eq)
        tgt = jnp.where(md, spk_base + defbase + cd,
                        jnp.where(ok_e, spk_base + n1 + rank_eq, trash))
        mst = md | me
        slot = _i32(AI_STGK) + cc * 16 + lanes
        plsc.store_scatter(ai, [slot], k, mask=mst)
        flatidx = tid * SHARD + lanes * SEG + step
        plsc.store_scatter(ai, [slot + (AI_STGV - AI_STGK)], flatidx, mask=mst)
        plsc.store_scatter(ai, [slot + (AI_STGT - AI_STGK)], tgt, mask=mst)
        return (cd + jnp.where(md, 1, 0), ce + jnp.where(me, 1, 0),
                cc + jnp.where(mst, 1, 0))
    _, _, cc_tot = lax.fori_loop(0, SEG, c2body, (zeros, zeros, zeros))
    ngroups = (jnp.max(cc_tot) + 7) // 8

    def gbody(g, _):
        pltpu.sync_copy(ai.at[pl.ds(AI_STGK + g * 128, 128)],
                        sp.at[ai.at[pl.ds(AI_STGT + g * 128, 128)]])
        for v8 in range(8):
            off = AI_STGT + g * 128 + v8 * 16
            ai[pl.ds(off, 16)] = ai[pl.ds(off, 16)] + (SP_SURVI - SP_SURVK)
        pltpu.sync_copy(ai.at[pl.ds(AI_STGV + g * 128, 128)],
                        sp.at[ai.at[pl.ds(AI_STGT + g * 128, 128)]])
        return 0
    lax.fori_loop(0, ngroups, gbody, 0)
    plsc.subcore_barrier()

    @pl.when(sid == 0)
    def _():
        pltpu.sync_copy(sp.at[pl.ds(spk_base, N1P)], ai.at[pl.ds(SO_SK, N1P)])
        pltpu.sync_copy(sp.at[pl.ds(SP_SURVI + cid * N1P, N1P)],
                        ai.at[pl.ds(SO_SV, N1P)])
        ai[pl.ds(SO_SK + K1, 16)] = zeros
        ai[pl.ds(SO_SV + K1, 16)] = zeros
        mink = _radix_sort(ai, SO_SK, SO_SV, SO_KB, SO_VB, N1, SEG2,
                           SO_H32A, SO_H32B, lanes, ones, zeros)

        pltpu.sync_copy(sp.at[pl.ds(SP_IDS + cid * 30720, 15360)],
                        ai.at[pl.ds(SO_SUB, 15360)])
        pltpu.sync_copy(sp.at[pl.ds(SP_IDS + cid * 30720 + 15360, 15360)],
                        ai.at[pl.ds(SO_OBJ, 15360)])
        pltpu.sync_copy(ent_hbm.at[cid], af.at[pl.ds(AF_ENT, ENTP)])

        def trip(ts, _):
            s = ts * 16 + lanes
            r = s // 3
            j = s - r * 3
            flatv = plsc.load_gather(ai, [_i32(SO_SV) + r])
            q = flatv // C
            kk = plsc.load_gather(ai, [_i32(SO_SK) + r])
            pb = plsc.bitcast(kk + mink, jnp.float32)
            sub = plsc.load_gather(ai, [_i32(SO_SUB) + q * 3 + j])
            obj = plsc.load_gather(ai, [_i32(SO_OBJ) + q * 3 + j])
            ssv = plsc.load_gather(af, [_i32(AF_ENT) + sub])
            osv = plsc.load_gather(af, [_i32(AF_ENT) + obj])
            trp = (pb * ssv) * osv
            ok = (s < K1 * 3) & (sub != obj)
            tkey = jnp.where(ok, plsc.bitcast(trp, jnp.int32) + 1, 0)
            ai[pl.ds(SO_TK + ts * 16, 16)] = tkey
            return 0
        lax.fori_loop(0, SEG3, trip, 0)

        t2 = jnp.int32(0)
        need2 = jnp.int32(K2)
        for lvl in range(4):
            sh = 24 - 8 * lvl

            def zh5(z, _):
                ai[pl.ds(SO_H2D5 + z * 16, 16)] = zeros
                return 0
            lax.fori_loop(0, 256, zh5, 0)

            def h5scan(step, _):
                k = plsc.load_gather(ai, [_i32(SO_TK) + lanes * SEG3 + step])
                d = lax.shift_right_logical(k, _i32(sh)) & 255
                hidx = _i32(SO_H2D5) + lanes * 256 + d
                if lvl == 0:
                    plsc.addupdate_scatter(ai, [hidx], ones)
                else:
                    m = lax.shift_right_logical(k, _i32(sh + 8)) == t2
                    plsc.addupdate_scatter(ai, [hidx], ones, mask=m)
                return 0
            lax.fori_loop(0, SEG3, h5scan, 0)

            def h5merge(j, _):
                acc = zeros
                for l in range(16):
                    acc = acc + ai[pl.ds(SO_H2D5 + l * 256 + j * 16, 16)]
                ai[pl.ds(SO_HM5 + j * 16, 16)] = acc
                return 0
            lax.fori_loop(0, 16, h5merge, 0)
            beta, sbeta = _suffix_find(ai, SO_HM5, need2, lanes)
            need2 = need2 - sbeta
            t2 = t2 * 256 + beta

        def c5body(step, car):
            cd, ce = car
            k = plsc.load_gather(ai, [_i32(SO_TK) + lanes * SEG3 + step])
            cd = cd + jnp.where(k > t2, 1, 0)
            ce = ce + jnp.where(k == t2, 1, 0)
            return cd, ce
        cd2, ce2 = lax.fori_loop(0, SEG3, c5body, (zeros, zeros))
        n12 = jnp.sum(cd2)
        defb2 = plsc.cumsum(cd2) - cd2
        eqb2 = plsc.cumsum(ce2) - ce2

        def c5place(step, car):
            cd, ce = car
            pos = lanes * SEG3 + step
            k = plsc.load_gather(ai, [_i32(SO_TK) + pos])
            md = k > t2
            me = k == t2
            rank_eq = eqb2 + ce
            ok_e = me & (rank_eq < need2)
            tgt = jnp.where(md, defb2 + cd, n12 + rank_eq)
            mst = md | ok_e
            plsc.store_scatter(ai, [_i32(SO_K2A) + tgt], k, mask=mst)
            plsc.store_scatter(ai, [_i32(SO_V2A) + tgt], pos, mask=mst)
            return cd + jnp.where(md, 1, 0), ce + jnp.where(me, 1, 0)
        lax.fori_loop(0, SEG3, c5place, (zeros, zeros))

        mink2 = _radix_sort(ai, SO_K2A, SO_V2A, SO_K2B, SO_V2B, K2, SEG4,
                            SO_H32A, SO_H32B, lanes, ones, zeros)
        del mink2

        def obody(os_, _):
            s = ai[pl.ds(SO_V2A + os_ * 16, 16)]
            r = s // 3
            j = s - r * 3
            flatv = plsc.load_gather(ai, [_i32(SO_SV) + r])
            q = flatv // C
            lab = flatv - q * C + 1
            kk = plsc.load_gather(ai, [_i32(SO_SK) + r])
            pb = plsc.bitcast(kk + mink, jnp.float32)
            sub = plsc.load_gather(ai, [_i32(SO_SUB) + q * 3 + j])
            obj = plsc.load_gather(ai, [_i32(SO_OBJ) + q * 3 + j])
            ssv = plsc.load_gather(af, [_i32(AF_ENT) + sub])
            osv = plsc.load_gather(af, [_i32(AF_ENT) + obj])
            trp = (pb * ssv) * osv
            rr6 = (os_ * 16 + lanes) * 6 + AF_OUT
            plsc.store_scatter(af, [rr6], sub.astype(jnp.float32))
            plsc.store_scatter(af, [rr6 + 1], obj.astype(jnp.float32))
            plsc.store_scatter(af, [rr6 + 2], lab.astype(jnp.float32))
            plsc.store_scatter(af, [rr6 + 3], trp)
            plsc.store_scatter(af, [rr6 + 4], pb)
            plsc.store_scatter(af, [rr6 + 5], q.astype(jnp.float32))
            return 0
        lax.fori_loop(0, SEG4, obody, 0)
        pltpu.sync_copy(af.at[pl.ds(AF_OUT, K2 * 6)], out_hbm.at[cid])


@functools.partial(
    pl.kernel,
    out_type=jax.ShapeDtypeStruct((2, K2 * 6), jnp.float32),
    mesh=plsc.VectorSubcoreMesh(core_axis_name="c", subcore_axis_name="s"),
    compiler_params=pltpu.CompilerParams(needs_layout_passes=False,
                                         use_tc_tiling_on_sc=True),
    scratch_types=[
        pltpu.VMEM((AI_TOTAL,), jnp.int32),
        pltpu.VMEM((AF_TOTAL,), jnp.float32),
        pltpu.VMEM_SHARED((SP_TOTAL,), jnp.int32),
    ],
)
def _sc_kernel(prob_hbm, smh_hbm, smo_hbm, tailh_hbm, tailo_hbm, ent_hbm,
               out_hbm, ai, af, sp):
    _sc_body(prob_hbm, smh_hbm, smo_hbm, tailh_hbm, tailo_hbm, ent_hbm,
             out_hbm, ai, af, sp)


def kernel(pred_rel_logits, pred_hidx, pred_oidx, ent_scores, ent_boxes,
           target_sizes):
    B = pred_rel_logits.shape[0]
    prob = jax.nn.sigmoid(pred_rel_logits).reshape(B, NQ * C)
    probf = jnp.pad(prob, ((0, 0), (0, NFLAT - NQ * C)))
    smh_p = jax.nn.softmax(pred_hidx, axis=-1).reshape(B, NQ * NENT)
    smo_p = jax.nn.softmax(pred_oidx, axis=-1).reshape(B, NQ * NENT)
    ent_p = jnp.pad(ent_scores, ((0, 0), (0, ENTP - NENT)))
    tail_h = jnp.pad(pred_hidx[:, RPB * 16:, :].reshape(B, 8 * NENT),
                     ((0, 0), (0, 2560 - 8 * NENT)))
    tail_o = jnp.pad(pred_oidx[:, RPB * 16:, :].reshape(B, 8 * NENT),
                     ((0, 0), (0, 2560 - 8 * NENT)))
    out = _sc_kernel(probf, smh_p, smo_p, tail_h, tail_o, ent_p)
    return out.reshape(B, K2, 6)
